# Initial kernel scaffold; baseline (speedup 1.0000x reference)
import jax
import jax.numpy as jnp
from jax.experimental import pallas as pl


def kernel(hidden_, target, target_path, target_path_len, target_code, embed_table):
    raise NotImplementedError("write your pallas kernel here")


# speedup vs baseline: 17.6514x; 17.6514x over previous
import functools

import jax
import jax.numpy as jnp
from jax import lax
from jax.experimental import pallas as pl
from jax.experimental.pallas import tpu as pltpu
from jax.experimental.pallas import tpu_sc as plsc

DIM = 128
B = 4096
L = 20
NW = 32
BW = B // NW
KW = BW * L
CHUNK = 128
NCHUNK = KW // CHUNK

_MESH = plsc.VectorSubcoreMesh(core_axis_name="c", subcore_axis_name="s")

_GATHER_DNUMS = lax.GatherDimensionNumbers(
    offset_dims=(), collapsed_slice_dims=(0,), start_index_map=(0,))


def _shfl(v, idx):
    return lax.gather(v, idx[:, None], _GATHER_DNUMS, slice_sizes=(1,),
                      mode=lax.GatherScatterMode.PROMISE_IN_BOUNDS)


@functools.partial(
    pl.kernel,
    out_type=jax.ShapeDtypeStruct((NW, KW), jnp.float32),
    mesh=_MESH,
    scratch_types=[
        pltpu.VMEM((NCHUNK, CHUNK), jnp.int32),
        pltpu.VMEM((CHUNK, DIM), jnp.float32),
        pltpu.VMEM((BW, DIM), jnp.float32),
        pltpu.VMEM((KW,), jnp.float32),
        pltpu.SemaphoreType.DMA,
    ],
)
def _sc_logits(table_hbm, tp_hbm, hid_hbm, out_hbm, idx_v, rows_v, hid_v, z_v, sem):
    wid = lax.axis_index("c") * 16 + lax.axis_index("s")
    pltpu.sync_copy(tp_hbm.at[wid], idx_v)
    pltpu.sync_copy(hid_hbm.at[wid], hid_v)
    lanes = lax.iota(jnp.int32, 16)

    def chunk_body(j, _):
        pltpu.async_copy(table_hbm.at[idx_v.at[j]], rows_v, sem).wait()

        def blk_body(t, _):
            k0 = j * CHUNK + t * 16
            zvec = jnp.zeros((16,), jnp.float32)
            for u in range(16):
                r = t * 16 + u
                b = (k0 + u) // L
                acc = rows_v[r, pl.ds(0, 16)] * hid_v[b, pl.ds(0, 16)]
                for s in range(1, DIM // 16):
                    acc = acc + rows_v[r, pl.ds(s * 16, 16)] * hid_v[b, pl.ds(s * 16, 16)]
                for sh in (8, 4, 2, 1):
                    acc = acc + _shfl(acc, jnp.bitwise_xor(lanes, sh))
                zvec = jnp.where(lanes == u, acc, zvec)
            z_v[pl.ds(k0, 16)] = zvec
            return 0

        lax.fori_loop(0, CHUNK // 16, blk_body, 0)
        return 0

    lax.fori_loop(0, NCHUNK, chunk_body, 0)
    pltpu.sync_copy(z_v, out_hbm.at[wid])


def _tc_loss_body(z_ref, code_ref, len_ref, out_ref):
    z = z_ref[...]
    flag = code_ref[...].astype(jnp.float32)
    x = z * (1.0 - 2.0 * flag)
    loss = jnp.maximum(x, 0.0) + jnp.log(1.0 + jnp.exp(-jnp.abs(x)))
    pos = lax.broadcasted_iota(jnp.int32, z.shape, 1)
    mask = (pos < len_ref[...]).astype(jnp.float32)
    out_ref[...] = (jnp.sum(loss * mask) / jnp.sum(mask)).reshape(1, 1)


_tc_loss = pl.pallas_call(
    _tc_loss_body,
    out_shape=jax.ShapeDtypeStruct((1, 1), jnp.float32),
)


def kernel(hidden_, target, target_path, target_path_len, target_code, embed_table):
    tp = target_path.reshape(NW, NCHUNK, CHUNK)
    hid = hidden_.reshape(NW, BW, DIM)
    z = _sc_logits(embed_table, tp, hid).reshape(B, L)
    loss = _tc_loss(z, target_code, target_path_len.reshape(B, 1))
    return loss[0, 0]


# speedup vs baseline: 23.2498x; 1.3172x over previous
import functools

import jax
import jax.numpy as jnp
from jax import lax
from jax.experimental import pallas as pl
from jax.experimental.pallas import tpu as pltpu
from jax.experimental.pallas import tpu_sc as plsc

DIM = 128
B = 4096
L = 20
NW = 32
BW = B // NW
KW = BW * L
CHUNK = 128
NCHUNK = KW // CHUNK

_MESH = plsc.VectorSubcoreMesh(core_axis_name="c", subcore_axis_name="s")

_GATHER_DNUMS = lax.GatherDimensionNumbers(
    offset_dims=(), collapsed_slice_dims=(0,), start_index_map=(0,))


def _shfl(v, idx):
    return lax.gather(v, idx[:, None], _GATHER_DNUMS, slice_sizes=(1,),
                      mode=lax.GatherScatterMode.PROMISE_IN_BOUNDS)


@functools.partial(
    pl.kernel,
    out_type=jax.ShapeDtypeStruct((NW, KW), jnp.float32),
    mesh=_MESH,
    scratch_types=[
        pltpu.VMEM((NCHUNK, CHUNK), jnp.int32),
        pltpu.VMEM((CHUNK, DIM), jnp.float32),
        pltpu.VMEM((CHUNK, DIM), jnp.float32),
        pltpu.VMEM((BW, DIM), jnp.float32),
        pltpu.VMEM((KW,), jnp.float32),
        pltpu.SemaphoreType.DMA,
        pltpu.SemaphoreType.DMA,
    ],
)
def _sc_logits(table_hbm, tp_hbm, hid_hbm, out_hbm,
               idx_v, rows0_v, rows1_v, hid_v, z_v, sem0, sem1):
    wid = lax.axis_index("c") * 16 + lax.axis_index("s")
    pltpu.sync_copy(tp_hbm.at[wid], idx_v)
    pltpu.sync_copy(hid_hbm.at[wid], hid_v)
    lanes = lax.iota(jnp.int32, 16)

    def start_gather(c, buf, sem):
        cc = jnp.minimum(c, NCHUNK - 1)
        pltpu.async_copy(table_hbm.at[idx_v.at[cc]], buf, sem)

    def wait_gather(buf, sem):
        pltpu.make_async_copy(table_hbm.at[idx_v.at[0]], buf, sem).wait()

    def compute(c, buf):
        def blk_body(t, _):
            k0 = c * CHUNK + t * 16
            b0 = k0 // L
            b1 = (k0 + 15) // L
            ub = b1 * L - k0
            h0 = [hid_v[b0, pl.ds(s * 16, 16)] for s in range(DIM // 16)]
            h1 = [hid_v[b1, pl.ds(s * 16, 16)] for s in range(DIM // 16)]
            zvec = jnp.zeros((16,), jnp.float32)
            for u in range(16):
                r = t * 16 + u
                in_b0 = u < ub
                acc = None
                for s in range(DIM // 16):
                    h = jnp.where(in_b0, h0[s], h1[s])
                    prod = buf[r, pl.ds(s * 16, 16)] * h
                    acc = prod if acc is None else acc + prod
                for sh in (8, 4, 2, 1):
                    acc = acc + _shfl(acc, jnp.bitwise_xor(lanes, sh))
                zvec = jnp.where(lanes == u, acc, zvec)
            z_v[pl.ds(k0, 16)] = zvec
            return 0

        lax.fori_loop(0, CHUNK // 16, blk_body, 0)

    start_gather(0, rows0_v, sem0)

    def chunk2_body(jj, _):
        c0 = 2 * jj
        wait_gather(rows0_v, sem0)
        start_gather(c0 + 1, rows1_v, sem1)
        compute(c0, rows0_v)
        wait_gather(rows1_v, sem1)
        start_gather(c0 + 2, rows0_v, sem0)
        compute(c0 + 1, rows1_v)
        return 0

    lax.fori_loop(0, NCHUNK // 2, chunk2_body, 0)
    wait_gather(rows0_v, sem0)
    pltpu.sync_copy(z_v, out_hbm.at[wid])


def _tc_loss_body(z_ref, code_ref, len_ref, out_ref):
    z = z_ref[...]
    flag = code_ref[...].astype(jnp.float32)
    x = z * (1.0 - 2.0 * flag)
    loss = jnp.maximum(x, 0.0) + jnp.log(1.0 + jnp.exp(-jnp.abs(x)))
    pos = lax.broadcasted_iota(jnp.int32, z.shape, 1)
    mask = (pos < len_ref[...]).astype(jnp.float32)
    out_ref[...] = (jnp.sum(loss * mask) / jnp.sum(mask)).reshape(1, 1)


_tc_loss = pl.pallas_call(
    _tc_loss_body,
    out_shape=jax.ShapeDtypeStruct((1, 1), jnp.float32),
)


def kernel(hidden_, target, target_path, target_path_len, target_code, embed_table):
    tp = target_path.reshape(NW, NCHUNK, CHUNK)
    hid = hidden_.reshape(NW, BW, DIM)
    z = _sc_logits(embed_table, tp, hid).reshape(B, L)
    loss = _tc_loss(z, target_code, target_path_len.reshape(B, 1))
    return loss[0, 0]


# speedup vs baseline: 24.7448x; 1.0643x over previous
import functools

import jax
import jax.numpy as jnp
from jax import lax
from jax.experimental import pallas as pl
from jax.experimental.pallas import tpu as pltpu
from jax.experimental.pallas import tpu_sc as plsc

DIM = 128
B = 4096
L = 20
NW = 32
BW = B // NW
KW = BW * L
CHUNK = 128
NCHUNK = KW // CHUNK
SENTINEL = -1e30

_MESH = plsc.VectorSubcoreMesh(core_axis_name="c", subcore_axis_name="s")

_GATHER_DNUMS = lax.GatherDimensionNumbers(
    offset_dims=(), collapsed_slice_dims=(0,), start_index_map=(0,))


def _shfl(v, idx):
    return lax.gather(v, idx[:, None], _GATHER_DNUMS, slice_sizes=(1,),
                      mode=lax.GatherScatterMode.PROMISE_IN_BOUNDS)


@functools.partial(
    pl.kernel,
    out_type=jax.ShapeDtypeStruct((NW, KW), jnp.float32),
    mesh=_MESH,
    scratch_types=[
        pltpu.VMEM((NCHUNK, CHUNK), jnp.int32),
        pltpu.VMEM((CHUNK, DIM), jnp.float32),
        pltpu.VMEM((CHUNK, DIM), jnp.float32),
        pltpu.VMEM((BW, DIM), jnp.float32),
        pltpu.VMEM((KW,), jnp.int32),
        pltpu.VMEM((BW + 16,), jnp.int32),
        pltpu.VMEM((KW,), jnp.float32),
        pltpu.SemaphoreType.DMA,
        pltpu.SemaphoreType.DMA,
    ],
)
def _sc_logits(table_hbm, tp_hbm, hid_hbm, code_hbm, len_hbm, out_hbm,
               idx_v, rows0_v, rows1_v, hid_v, code_v, len_v, z_v, sem0, sem1):
    wid = lax.axis_index("c") * 16 + lax.axis_index("s")
    base = wid * BW
    pltpu.sync_copy(tp_hbm.at[wid], idx_v)
    pltpu.sync_copy(hid_hbm.at[wid], hid_v)
    pltpu.sync_copy(code_hbm.at[wid], code_v)
    pltpu.sync_copy(len_hbm.at[wid], len_v.at[pl.ds(0, BW)])
    lanes = lax.iota(jnp.int32, 16)

    def start_gather(c, buf, sem):
        cc = jnp.minimum(c, NCHUNK - 1)
        pltpu.async_copy(table_hbm.at[idx_v.at[cc]], buf, sem)

    def wait_gather(buf, sem):
        pltpu.make_async_copy(table_hbm.at[idx_v.at[0]], buf, sem).wait()

    def compute(c, buf):
        def blk_body(t, _):
            k0 = c * CHUNK + t * 16
            b0 = k0 // L
            b1 = (k0 + 15) // L
            ub = b1 * L - k0
            h0 = [hid_v[b0, pl.ds(s * 16, 16)] for s in range(DIM // 16)]
            h1 = [hid_v[b1, pl.ds(s * 16, 16)] for s in range(DIM // 16)]
            zvec = jnp.zeros((16,), jnp.float32)
            for u in range(16):
                r = t * 16 + u
                in_b0 = u < ub
                acc = None
                for s in range(DIM // 16):
                    h = jnp.where(in_b0, h0[s], h1[s])
                    prod = buf[r, pl.ds(s * 16, 16)] * h
                    acc = prod if acc is None else acc + prod
                for sh in (8, 4, 2, 1):
                    acc = acc + _shfl(acc, jnp.bitwise_xor(lanes, sh))
                zvec = jnp.where(lanes == u, acc, zvec)
            lvec = (k0 % L) + lanes
            lvec = jnp.where(lvec >= L, lvec - L, lvec)
            len0 = len_v[pl.ds(b0, 16)][0]
            len1 = len_v[pl.ds(b1, 16)][0]
            lenv = jnp.where(lanes < ub, len0, len1)
            codev = code_v[pl.ds(k0, 16)]
            sign = 1.0 - 2.0 * codev.astype(jnp.float32)
            x = jnp.where(lvec < lenv, zvec * sign, jnp.float32(SENTINEL))
            z_v[pl.ds(k0, 16)] = x
            return 0

        lax.fori_loop(0, CHUNK // 16, blk_body, 0)

    start_gather(0, rows0_v, sem0)

    def chunk2_body(jj, _):
        c0 = 2 * jj
        wait_gather(rows0_v, sem0)
        start_gather(c0 + 1, rows1_v, sem1)
        compute(c0, rows0_v)
        wait_gather(rows1_v, sem1)
        start_gather(c0 + 2, rows0_v, sem0)
        compute(c0 + 1, rows1_v)
        return 0

    lax.fori_loop(0, NCHUNK // 2, chunk2_body, 0)
    wait_gather(rows0_v, sem0)
    pltpu.sync_copy(z_v, out_hbm.at[wid])


def _tc_loss_body(x_ref, out_ref):
    x = x_ref[...]
    loss = jnp.maximum(x, 0.0) + jnp.log(1.0 + jnp.exp(-jnp.abs(x)))
    cnt = jnp.sum((x > SENTINEL * 0.5).astype(jnp.float32))
    out_ref[...] = (jnp.sum(loss) / cnt).reshape(1, 1)


_tc_loss = pl.pallas_call(
    _tc_loss_body,
    out_shape=jax.ShapeDtypeStruct((1, 1), jnp.float32),
)


def kernel(hidden_, target, target_path, target_path_len, target_code, embed_table):
    tp = target_path.reshape(NW, NCHUNK, CHUNK)
    code = target_code.reshape(NW, KW)
    hid = hidden_.reshape(NW, BW, DIM)
    x = _sc_logits(embed_table, tp, hid, code, target_path_len.reshape(NW, BW))
    loss = _tc_loss(x)
    return loss[0, 0]


# speedup vs baseline: 25.0471x; 1.0122x over previous
import functools

import jax
import jax.numpy as jnp
from jax import lax
from jax.experimental import pallas as pl
from jax.experimental.pallas import tpu as pltpu
from jax.experimental.pallas import tpu_sc as plsc

DIM = 128
B = 4096
L = 20
NW = 32
BW = B // NW
KW = BW * L
CHUNK = 128
NCHUNK = KW // CHUNK
SENTINEL = -1e30

_MESH = plsc.VectorSubcoreMesh(core_axis_name="c", subcore_axis_name="s")

_GATHER_DNUMS = lax.GatherDimensionNumbers(
    offset_dims=(), collapsed_slice_dims=(0,), start_index_map=(0,))


def _shfl(v, idx):
    return lax.gather(v, idx[:, None], _GATHER_DNUMS, slice_sizes=(1,),
                      mode=lax.GatherScatterMode.PROMISE_IN_BOUNDS)


@functools.partial(
    pl.kernel,
    out_type=jax.ShapeDtypeStruct((NW, KW), jnp.float32),
    mesh=_MESH,
    scratch_types=[
        pltpu.VMEM((NCHUNK, CHUNK), jnp.int32),
        pltpu.VMEM((2 * CHUNK, DIM), jnp.float32),
        pltpu.VMEM((BW, DIM), jnp.float32),
        pltpu.VMEM((KW,), jnp.int32),
        pltpu.VMEM((BW + 16,), jnp.int32),
        pltpu.VMEM((KW,), jnp.float32),
        pltpu.SemaphoreType.DMA,
        pltpu.SemaphoreType.DMA,
    ],
)
def _sc_logits(table_hbm, tp_hbm, hid_hbm, code_hbm, len_hbm, out_hbm,
               idx_v, rows_v, hid_v, code_v, len_v, z_v, sem0, sem1):
    wid = lax.axis_index("c") * 16 + lax.axis_index("s")
    base = wid * BW
    pltpu.sync_copy(tp_hbm.at[wid], idx_v)
    pltpu.sync_copy(hid_hbm.at[pl.ds(base, BW)], hid_v)
    pltpu.sync_copy(code_hbm.at[wid], code_v)
    pltpu.sync_copy(len_hbm.at[pl.ds(base, BW)], len_v.at[pl.ds(0, BW)])
    lanes = lax.iota(jnp.int32, 16)

    half0, half1 = rows_v.at[pl.ds(0, CHUNK)], rows_v.at[pl.ds(CHUNK, CHUNK)]

    def start_gather(c, buf, sem):
        cc = jnp.minimum(c, NCHUNK - 1)
        pltpu.async_copy(table_hbm.at[idx_v.at[cc]], buf, sem)

    def wait_gather(buf, sem):
        pltpu.make_async_copy(table_hbm.at[idx_v.at[0]], buf, sem).wait()

    def compute(c, off):
        def blk_body(t, _):
            k0 = c * CHUNK + t * 16
            b0 = k0 // L
            b1 = (k0 + 15) // L
            ub = b1 * L - k0
            h0 = [hid_v[b0, pl.ds(s * 16, 16)] for s in range(DIM // 16)]
            h1 = [hid_v[b1, pl.ds(s * 16, 16)] for s in range(DIM // 16)]
            zvec = jnp.zeros((16,), jnp.float32)
            for u in range(16):
                r = t * 16 + u
                in_b0 = u < ub
                acc = None
                for s in range(DIM // 16):
                    h = jnp.where(in_b0, h0[s], h1[s])
                    prod = rows_v[off + r, pl.ds(s * 16, 16)] * h
                    acc = prod if acc is None else acc + prod
                for sh in (8, 4, 2, 1):
                    acc = acc + _shfl(acc, jnp.bitwise_xor(lanes, sh))
                zvec = jnp.where(lanes == u, acc, zvec)
            lvec = (k0 % L) + lanes
            lvec = jnp.where(lvec >= L, lvec - L, lvec)
            len0 = len_v[pl.ds(b0, 16)][0]
            len1 = len_v[pl.ds(b1, 16)][0]
            lenv = jnp.where(lanes < ub, len0, len1)
            codev = code_v[pl.ds(k0, 16)]
            sign = 1.0 - 2.0 * codev.astype(jnp.float32)
            x = jnp.where(lvec < lenv, zvec * sign, jnp.float32(SENTINEL))
            z_v[pl.ds(k0, 16)] = x
            return 0

        lax.fori_loop(0, CHUNK // 16, blk_body, 0)

    start_gather(0, half0, sem0)

    def chunk_body(c, _):
        even = c % 2 == 0

        @pl.when(even)
        def _():
            wait_gather(half0, sem0)
            start_gather(c + 1, half1, sem1)

        @pl.when(jnp.logical_not(even))
        def _():
            wait_gather(half1, sem1)
            start_gather(c + 1, half0, sem0)

        compute(c, (c % 2) * CHUNK)
        return 0

    lax.fori_loop(0, NCHUNK, chunk_body, 0)
    wait_gather(half0, sem0)
    pltpu.sync_copy(z_v, out_hbm.at[wid])


def _tc_loss_body(x_ref, out_ref):
    x = x_ref[...]
    loss = jnp.maximum(x, 0.0) + jnp.log(1.0 + jnp.exp(-jnp.abs(x)))
    cnt = jnp.sum((x > SENTINEL * 0.5).astype(jnp.float32))
    out_ref[...] = (jnp.sum(loss) / cnt).reshape(1, 1)


_tc_loss = pl.pallas_call(
    _tc_loss_body,
    out_shape=jax.ShapeDtypeStruct((1, 1), jnp.float32),
)


def kernel(hidden_, target, target_path, target_path_len, target_code, embed_table):
    tp = target_path.reshape(NW, NCHUNK, CHUNK)
    code = target_code.reshape(NW, KW)
    x = _sc_logits(embed_table, tp, hidden_, code, target_path_len)
    loss = _tc_loss(x)
    return loss[0, 0]


# speedup vs baseline: 25.2115x; 1.0066x over previous
import functools

import jax
import jax.numpy as jnp
from jax import lax
from jax.experimental import pallas as pl
from jax.experimental.pallas import tpu as pltpu
from jax.experimental.pallas import tpu_sc as plsc

DIM = 128
B = 4096
L = 20
NW = 32
BW = B // NW
KW = BW * L
CHUNK = 128
NCHUNK = KW // CHUNK
SENTINEL = -1e30

_MESH = plsc.VectorSubcoreMesh(core_axis_name="c", subcore_axis_name="s")

_GATHER_DNUMS = lax.GatherDimensionNumbers(
    offset_dims=(), collapsed_slice_dims=(0,), start_index_map=(0,))


def _shfl(v, idx):
    return lax.gather(v, idx[:, None], _GATHER_DNUMS, slice_sizes=(1,),
                      mode=lax.GatherScatterMode.PROMISE_IN_BOUNDS)


@functools.partial(
    pl.kernel,
    out_type=jax.ShapeDtypeStruct((NW, KW), jnp.float32),
    mesh=_MESH,
    scratch_types=[
        pltpu.VMEM((NCHUNK, CHUNK), jnp.int32),
        pltpu.VMEM((2 * CHUNK, DIM), jnp.float32),
        pltpu.VMEM((BW, DIM), jnp.float32),
        pltpu.VMEM((BW, L), jnp.int32),
        pltpu.VMEM((BW + 16,), jnp.int32),
        pltpu.VMEM((KW,), jnp.float32),
        pltpu.SemaphoreType.DMA,
        pltpu.SemaphoreType.DMA,
    ],
)
def _sc_logits(table_hbm, tp_hbm, hid_hbm, code_hbm, len_hbm, out_hbm,
               idx_v, rows_v, hid_v, code_v, len_v, z_v, sem0, sem1):
    wid = lax.axis_index("c") * 16 + lax.axis_index("s")
    base = wid * BW
    pltpu.sync_copy(tp_hbm.at[wid], idx_v)
    pltpu.sync_copy(hid_hbm.at[pl.ds(base, BW)], hid_v)
    pltpu.sync_copy(code_hbm.at[pl.ds(base, BW)], code_v)
    pltpu.sync_copy(len_hbm.at[pl.ds(base, BW)], len_v.at[pl.ds(0, BW)])
    lanes = lax.iota(jnp.int32, 16)

    half0, half1 = rows_v.at[pl.ds(0, CHUNK)], rows_v.at[pl.ds(CHUNK, CHUNK)]

    def start_gather(c, buf, sem):
        cc = jnp.minimum(c, NCHUNK - 1)
        pltpu.async_copy(table_hbm.at[idx_v.at[cc]], buf, sem)

    def wait_gather(buf, sem):
        pltpu.make_async_copy(table_hbm.at[idx_v.at[0]], buf, sem).wait()

    def compute(c, off):
        def blk_body(t, _):
            k0 = c * CHUNK + t * 16
            b0 = k0 // L
            b1 = (k0 + 15) // L
            l0 = k0 - b0 * L
            ub = b1 * L - k0
            h0 = [hid_v[b0, pl.ds(s * 16, 16)] for s in range(DIM // 16)]
            h1 = [hid_v[b1, pl.ds(s * 16, 16)] for s in range(DIM // 16)]
            zvec = jnp.zeros((16,), jnp.float32)
            for u in range(16):
                r = t * 16 + u
                in_b0 = u < ub
                acc = None
                for s in range(DIM // 16):
                    h = jnp.where(in_b0, h0[s], h1[s])
                    prod = rows_v[off + r, pl.ds(s * 16, 16)] * h
                    acc = prod if acc is None else acc + prod
                for sh in (8, 4, 2, 1):
                    acc = acc + _shfl(acc, jnp.bitwise_xor(lanes, sh))
                zvec = jnp.where(lanes == u, acc, zvec)
            lvec = l0 + lanes
            lvec = jnp.where(lvec >= L, lvec - L, lvec)
            len0 = len_v[pl.ds(b0, 16)][0]
            len1 = len_v[pl.ds(b1, 16)][0]
            lenv = jnp.where(lanes < ub, len0, len1)
            st0 = jnp.minimum(l0, L - 16)
            code0 = _shfl(code_v[b0, pl.ds(st0, 16)],
                          jnp.minimum(lanes + (l0 - st0), 15))
            code1 = _shfl(code_v[b1, pl.ds(0, 16)],
                          jnp.maximum(lanes - ub, 0))
            codev = jnp.where(lanes < ub, code0, code1)
            sign = 1.0 - 2.0 * codev.astype(jnp.float32)
            x = jnp.where(lvec < lenv, zvec * sign, jnp.float32(SENTINEL))
            z_v[pl.ds(k0, 16)] = x
            return 0

        lax.fori_loop(0, CHUNK // 16, blk_body, 0)

    start_gather(0, half0, sem0)

    def chunk_body(c, _):
        even = c % 2 == 0

        @pl.when(even)
        def _():
            wait_gather(half0, sem0)
            start_gather(c + 1, half1, sem1)

        @pl.when(jnp.logical_not(even))
        def _():
            wait_gather(half1, sem1)
            start_gather(c + 1, half0, sem0)

        compute(c, (c % 2) * CHUNK)
        return 0

    lax.fori_loop(0, NCHUNK, chunk_body, 0)
    wait_gather(half0, sem0)
    pltpu.sync_copy(z_v, out_hbm.at[wid])


def _tc_loss_body(x_ref, out_ref):
    x = x_ref[...]
    loss = jnp.maximum(x, 0.0) + jnp.log(1.0 + jnp.exp(-jnp.abs(x)))
    cnt = jnp.sum((x > SENTINEL * 0.5).astype(jnp.float32))
    out_ref[...] = (jnp.sum(loss) / cnt).reshape(1, 1)


_tc_loss = pl.pallas_call(
    _tc_loss_body,
    out_shape=jax.ShapeDtypeStruct((1, 1), jnp.float32),
)


def kernel(hidden_, target, target_path, target_path_len, target_code, embed_table):
    tp = target_path.reshape(NW, NCHUNK, CHUNK)
    x = _sc_logits(embed_table, tp, hidden_, target_code, target_path_len)
    loss = _tc_loss(x)
    return loss[0, 0]
